# baseline (speedup 1.0000x reference)
import functools

import jax
import jax.numpy as jnp
import numpy as np
from jax import lax
from jax.experimental import pallas as pl
from jax.experimental.pallas import tpu as pltpu
from jax.experimental.pallas import tpu_sc as plsc

_SCALE_RANGES = ((1, 96), (48, 192), (96, 384), (192, 768), (384, 2048))
_FPN_SIZE = (40, 36, 24, 16, 12)
_IMG = 512
_SIG = 0.1
_NOBJ = 32
_MPB = 8

_NCELL = sum(s * s for s in _FPN_SIZE)
_NCHUNK_IMG = _NCELL // 16
_NCHUNK = 4 * _NCHUNK_IMG
_NWORK = 32
_CPW = 31
_NOUT = _NWORK * _CPW * 16

_LVL_CHUNK_BASE = tuple(np.cumsum([0] + [s * s // 16 for s in _FPN_SIZE[:-1]]).tolist())
_RTAB = np.concatenate(
    [np.repeat(np.arange(s), s) for s in _FPN_SIZE]).astype(np.float32)
_CTAB = np.concatenate(
    [np.tile(np.arange(s), s) for s in _FPN_SIZE]).astype(np.float32)


def _stats_body(mask_ref, boxes_ref, labels_ref, tab_ref, acc_ref):
    si = lax.broadcasted_iota(jnp.int32, (8, _IMG), 0)
    ki = lax.broadcasted_iota(jnp.int32, (8, _IMG), 1)
    w = jnp.where(si == 0, 1,
                  jnp.where(si == 1, ki & ~1,
                            jnp.where(si == 2, ki & 1, 0))).astype(jnp.float32)
    ci = lax.broadcasted_iota(jnp.int32, (1, _IMG), 1)
    lane = lax.broadcasted_iota(jnp.int32, (8, 128), 1)
    row = lax.broadcasted_iota(jnp.int32, (8, 128), 0)
    o_step = pl.program_id(1)
    o_base = o_step * _MPB
    acc = acc_ref[...]
    for j in range(_MPB):
        m = mask_ref[0, j]
        ind = jnp.where(m != 0.0, jnp.float32(1.0), jnp.float32(0.0))
        r = lax.dot_general(w, ind, (((1,), (0,)), ((), ())),
                            preferred_element_type=jnp.float32)
        per_col = r[0:1, :].astype(jnp.int32)
        count = jnp.sum(per_col)
        colsum = jnp.sum(per_col * ci)
        rowsum = (jnp.sum(r[1:2, :].astype(jnp.int32))
                  + jnp.sum(r[2:3, :].astype(jnp.int32)))
        hit = lane == o_base + j
        acc = jnp.where(hit & (row == 0), count,
                        jnp.where(hit & (row == 1), rowsum,
                                  jnp.where(hit & (row == 2), colsum, acc)))
    acc_ref[...] = acc

    @pl.when(o_step == pl.num_programs(1) - 1)
    def _finalize():
        cnt = acc[0:1, :]
        rsum = acc[1:2, :]
        csum = acc[2:3, :]
        b = boxes_ref[0]
        x1, y1, x2, y2 = b[0:1, :], b[1:2, :], b[2:3, :], b[3:4, :]
        labs = labels_ref[0].astype(jnp.float32)
        one = jnp.float32(1.0)
        hl = y2 - y1 + one
        wl = x2 - x1 + one
        area = jnp.sqrt(hl * wl)
        safe = jnp.maximum(cnt.astype(jnp.float32), one)
        has = cnt > 0
        half = jnp.float32(0.5)
        y_mean = jnp.where(has, rsum.astype(jnp.float32) / safe, half * (y1 + y2))
        x_mean = jnp.where(has, csum.astype(jnp.float32) / safe, half * (x1 + x2))
        sig = jnp.float32(_SIG)
        lim = jnp.float32(_IMG - 1)
        zero = jnp.float32(0.0)
        left = jnp.clip(x_mean - sig * wl, zero, lim)
        right = jnp.clip(x_mean + sig * wl, zero, lim)
        top = jnp.clip(y_mean - sig * hl, zero, lim)
        bot = jnp.clip(y_mean + sig * hl, zero, lim)
        inf = jnp.float32(jnp.inf)
        rowi = lax.broadcasted_iota(jnp.int32, (32, 128), 0)
        lanei = lax.broadcasted_iota(jnp.int32, (1, 128), 1)
        valid = lanei < _NOBJ
        tab = jnp.zeros((32, 128), jnp.float32)
        for i, s in enumerate(_FPN_SIZE):
            scale = jnp.float32(_IMG / s)
            lo, hi = _SCALE_RANGES[i]
            in_r = (area >= jnp.float32(lo)) & (area <= jnp.float32(hi))
            smax = jnp.float32(s - 1)
            p_l = jnp.clip(jnp.floor(left / scale), zero, smax)
            p_r = jnp.clip(jnp.floor(right / scale), zero, smax)
            p_t = jnp.clip(jnp.floor(top / scale), zero, smax)
            p_b = jnp.clip(jnp.floor(bot / scale), zero, smax)
            keyv = jnp.where(in_r, area, inf)
            tab = jnp.where(rowi == 4 * i + 0, p_l, tab)
            tab = jnp.where(rowi == 4 * i + 1, p_r, tab)
            tab = jnp.where(rowi == 4 * i + 2, p_t, tab)
            tab = jnp.where(rowi == 4 * i + 3, p_b, tab)
            tab = jnp.where(rowi == 20 + i, keyv, tab)
            anyflag = jnp.max(jnp.where(in_r & valid, jnp.float32(1.0), zero))
            tab = jnp.where(rowi == 26 + i, anyflag, tab)
        tab = jnp.where(rowi == 25, labs, tab)
        tab_ref[0] = tab


def _sc_assign(tab_hbm, rtab_hbm, ctab_hbm, cat_hbm, pt_hbm,
               tab_v, rtab_v, ctab_v, cat_st, pt_st):
    wid = lax.axis_index("s") * 2 + lax.axis_index("c")
    pltpu.sync_copy(tab_hbm, tab_v)
    pltpu.sync_copy(rtab_hbm, rtab_v)
    pltpu.sync_copy(ctab_hbm, ctab_v)
    inf = jnp.float32(jnp.inf)
    base_chunk = wid * _CPW

    def chunk_body(t, carry):
        g = jnp.minimum(base_chunk + t, _NCHUNK - 1)
        img = g // _NCHUNK_IMG
        cib = g % _NCHUNK_IMG
        lvl = ((cib >= _LVL_CHUNK_BASE[1]).astype(jnp.int32)
               + (cib >= _LVL_CHUNK_BASE[2]).astype(jnp.int32)
               + (cib >= _LVL_CHUNK_BASE[3]).astype(jnp.int32)
               + (cib >= _LVL_CHUNK_BASE[4]).astype(jnp.int32))
        rf = rtab_v[pl.ds(cib * 16, 16)]
        cf = ctab_v[pl.ds(cib * 16, 16)]
        tbase = img * 4096 + lvl * 4 * 128
        kbase = img * 4096 + (20 + lvl) * 128
        lbase = img * 4096 + 25 * 128
        fbase = img * 4096 + (26 + lvl) * 128
        flagv = tab_v[pl.ds(fbase, 16)]
        has_any = lax.squeeze(lax.slice(flagv, (0,), (1,)), (0,)) > 0.0

        def _do(_):
            quads = []
            for off in (0, 128, 256, 384):
                quads.append((tab_v[pl.ds(tbase + off, 16)],
                              tab_v[pl.ds(tbase + off + 16, 16)]))
            key_lo = tab_v[pl.ds(kbase, 16)]
            key_hi = tab_v[pl.ds(kbase + 16, 16)]
            lab_lo = tab_v[pl.ds(lbase, 16)]
            lab_hi = tab_v[pl.ds(lbase + 16, 16)]
            best = jnp.full((16,), inf, jnp.float32)
            bpt = jnp.full((16,), -1, jnp.int32)
            bcat = jnp.zeros((16,), jnp.int32)
            for o in range(_NOBJ):
                half_idx = o % 16

                def pick(lo, hi):
                    v = lo if o < 16 else hi
                    s1 = lax.slice(v, (half_idx,), (half_idx + 1,))
                    return jnp.broadcast_to(s1, (16,))

                p_l = pick(*quads[0])
                p_r = pick(*quads[1])
                p_t = pick(*quads[2])
                p_b = pick(*quads[3])
                key = pick(key_lo, key_hi)
                lab = pick(lab_lo, lab_hi)
                rect = (rf >= p_t) & (rf <= p_b) & (cf >= p_l) & (cf <= p_r)
                infv = jnp.full((16,), inf, jnp.float32)
                keyrect = jnp.where(rect, key, infv)
                upd = (keyrect <= best) & (keyrect < infv)
                best = jnp.where(upd, keyrect, best)
                bpt = jnp.where(upd, o, bpt)
                bcat = jnp.where(upd, lab.astype(jnp.int32), bcat)
            cat_st[pl.ds(t * 16, 16)] = bcat
            pt_st[pl.ds(t * 16, 16)] = bpt
            return 0

        def _skip(_):
            cat_st[pl.ds(t * 16, 16)] = jnp.zeros((16,), jnp.int32)
            pt_st[pl.ds(t * 16, 16)] = jnp.full((16,), -1, jnp.int32)
            return 0

        lax.cond(has_any, _do, _skip, 0)
        return carry

    lax.fori_loop(0, _CPW, chunk_body, 0)
    pltpu.sync_copy(cat_st, cat_hbm.at[pl.ds(base_chunk * 16, _CPW * 16)])
    pltpu.sync_copy(pt_st, pt_hbm.at[pl.ds(base_chunk * 16, _CPW * 16)])


def kernel(boxes, labels, masks):
    n, obj = masks.shape[0], masks.shape[1]
    boxes = jnp.asarray(boxes, dtype=jnp.float32)
    labels = jnp.asarray(labels, dtype=jnp.int32)
    masks = jnp.asarray(masks, dtype=jnp.float32)
    boxes_p = jnp.pad(boxes.transpose(0, 2, 1),
                      ((0, 0), (0, 4), (0, 128 - obj)))
    labels_p = jnp.pad(labels.reshape(n, 1, obj),
                       ((0, 0), (0, 0), (0, 128 - obj)))
    tab = pl.pallas_call(
        _stats_body,
        grid=(n, obj // _MPB),
        in_specs=[
            pl.BlockSpec((1, _MPB, _IMG, _IMG), lambda b, o: (b, o, 0, 0)),
            pl.BlockSpec((1, 8, 128), lambda b, o: (b, 0, 0)),
            pl.BlockSpec((1, 1, 128), lambda b, o: (b, 0, 0)),
        ],
        out_specs=pl.BlockSpec((1, 32, 128), lambda b, o: (b, 0, 0)),
        out_shape=jax.ShapeDtypeStruct((n, 32, 128), jnp.float32),
        scratch_shapes=[pltpu.VMEM((8, 128), jnp.int32)],
    )(masks, boxes_p, labels_p)
    tab_flat = tab.reshape(n * 32 * 128)
    mesh = plsc.VectorSubcoreMesh(core_axis_name="c", subcore_axis_name="s")
    sc = functools.partial(
        pl.kernel,
        mesh=mesh,
        out_type=[
            jax.ShapeDtypeStruct((_NOUT,), jnp.int32),
            jax.ShapeDtypeStruct((_NOUT,), jnp.int32),
        ],
        scratch_types=[
            pltpu.VMEM((n * 32 * 128,), jnp.float32),
            pltpu.VMEM((_NCELL,), jnp.float32),
            pltpu.VMEM((_NCELL,), jnp.float32),
            pltpu.VMEM((_CPW * 16,), jnp.int32),
            pltpu.VMEM((_CPW * 16,), jnp.int32),
        ],
    )(_sc_assign)
    cat_flat, pt_flat = sc(tab_flat, jnp.asarray(_RTAB), jnp.asarray(_CTAB))
    cat = cat_flat[:4 * _NCELL].reshape(n, _NCELL)
    pt = pt_flat[:4 * _NCELL].reshape(n, _NCELL)
    return cat, pt


# speedup vs baseline: 1.0105x; 1.0105x over previous
import functools

import jax
import jax.numpy as jnp
import numpy as np
from jax import lax
from jax.experimental import pallas as pl
from jax.experimental.pallas import tpu as pltpu
from jax.experimental.pallas import tpu_sc as plsc

_SCALE_RANGES = ((1, 96), (48, 192), (96, 384), (192, 768), (384, 2048))
_FPN_SIZE = (40, 36, 24, 16, 12)
_IMG = 512
_SIG = 0.1
_NOBJ = 32
_MPB = 8

_NCELL = sum(s * s for s in _FPN_SIZE)
_NCHUNK_IMG = _NCELL // 16
_NCHUNK = 4 * _NCHUNK_IMG
_NWORK = 32
_CPW = 31
_NOUT = _NWORK * _CPW * 16

_LVL_CHUNK_BASE = tuple(np.cumsum([0] + [s * s // 16 for s in _FPN_SIZE[:-1]]).tolist())
_RTAB = np.concatenate(
    [np.repeat(np.arange(s), s) for s in _FPN_SIZE]).astype(np.float32)
_CTAB = np.concatenate(
    [np.tile(np.arange(s), s) for s in _FPN_SIZE]).astype(np.float32)


def _stats_body(mask_ref, boxes_ref, labels_ref, tab_ref, acc_ref):
    si = lax.broadcasted_iota(jnp.int32, (8, _IMG), 0)
    ki = lax.broadcasted_iota(jnp.int32, (8, _IMG), 1)
    w = jnp.where(si == 0, 1,
                  jnp.where(si == 1, ki & ~1,
                            jnp.where(si == 2, ki & 1, 0))).astype(jnp.float32)
    ci = lax.broadcasted_iota(jnp.int32, (1, _IMG), 1)
    lane = lax.broadcasted_iota(jnp.int32, (8, 128), 1)
    row = lax.broadcasted_iota(jnp.int32, (8, 128), 0)
    o_step = pl.program_id(1)
    o_base = o_step * _MPB
    acc = acc_ref[...]
    for j in range(_MPB):
        m = mask_ref[0, j]
        ind = jnp.where(m != 0.0, jnp.float32(1.0), jnp.float32(0.0))
        r = lax.dot_general(w, ind, (((1,), (0,)), ((), ())),
                            preferred_element_type=jnp.float32)
        per_col = r[0:1, :].astype(jnp.int32)
        count = jnp.sum(per_col)
        colsum = jnp.sum(per_col * ci)
        rowsum = (jnp.sum(r[1:2, :].astype(jnp.int32))
                  + jnp.sum(r[2:3, :].astype(jnp.int32)))
        hit = lane == o_base + j
        acc = jnp.where(hit & (row == 0), count,
                        jnp.where(hit & (row == 1), rowsum,
                                  jnp.where(hit & (row == 2), colsum, acc)))
    acc_ref[...] = acc

    @pl.when(o_step == pl.num_programs(1) - 1)
    def _finalize():
        cnt = acc[0:1, :]
        rsum = acc[1:2, :]
        csum = acc[2:3, :]
        b = boxes_ref[0]
        x1, y1, x2, y2 = b[0:1, :], b[1:2, :], b[2:3, :], b[3:4, :]
        labs = labels_ref[0].astype(jnp.float32)
        one = jnp.float32(1.0)
        hl = y2 - y1 + one
        wl = x2 - x1 + one
        area = jnp.sqrt(hl * wl)
        safe = jnp.maximum(cnt.astype(jnp.float32), one)
        has = cnt > 0
        half = jnp.float32(0.5)
        y_mean = jnp.where(has, rsum.astype(jnp.float32) / safe, half * (y1 + y2))
        x_mean = jnp.where(has, csum.astype(jnp.float32) / safe, half * (x1 + x2))
        sig = jnp.float32(_SIG)
        lim = jnp.float32(_IMG - 1)
        zero = jnp.float32(0.0)
        left = jnp.clip(x_mean - sig * wl, zero, lim)
        right = jnp.clip(x_mean + sig * wl, zero, lim)
        top = jnp.clip(y_mean - sig * hl, zero, lim)
        bot = jnp.clip(y_mean + sig * hl, zero, lim)
        inf = jnp.float32(jnp.inf)
        rowi = lax.broadcasted_iota(jnp.int32, (32, 128), 0)
        tab = jnp.zeros((32, 128), jnp.float32)
        for i, s in enumerate(_FPN_SIZE):
            scale = jnp.float32(_IMG / s)
            lo, hi = _SCALE_RANGES[i]
            in_r = (area >= jnp.float32(lo)) & (area <= jnp.float32(hi))
            smax = jnp.float32(s - 1)
            p_l = jnp.clip(jnp.floor(left / scale), zero, smax)
            p_r = jnp.clip(jnp.floor(right / scale), zero, smax)
            p_t = jnp.clip(jnp.floor(top / scale), zero, smax)
            p_b = jnp.clip(jnp.floor(bot / scale), zero, smax)
            keyv = jnp.where(in_r, area, inf)
            tab = jnp.where(rowi == 4 * i + 0, p_l, tab)
            tab = jnp.where(rowi == 4 * i + 1, p_r, tab)
            tab = jnp.where(rowi == 4 * i + 2, p_t, tab)
            tab = jnp.where(rowi == 4 * i + 3, p_b, tab)
            tab = jnp.where(rowi == 20 + i, keyv, tab)
        tab = jnp.where(rowi == 25, labs, tab)
        tab_ref[0] = tab


def _sc_assign(tab_hbm, rtab_hbm, ctab_hbm, cat_hbm, pt_hbm,
               tab_v, rtab_v, ctab_v, cat_st, pt_st):
    wid = lax.axis_index("s") * 2 + lax.axis_index("c")
    pltpu.sync_copy(tab_hbm, tab_v)
    pltpu.sync_copy(rtab_hbm, rtab_v)
    pltpu.sync_copy(ctab_hbm, ctab_v)
    inf = jnp.float32(jnp.inf)
    base_chunk = wid * _CPW

    def chunk_body(t, carry):
        g = jnp.minimum(base_chunk + t, _NCHUNK - 1)
        img = g // _NCHUNK_IMG
        cib = g % _NCHUNK_IMG
        lvl = ((cib >= _LVL_CHUNK_BASE[1]).astype(jnp.int32)
               + (cib >= _LVL_CHUNK_BASE[2]).astype(jnp.int32)
               + (cib >= _LVL_CHUNK_BASE[3]).astype(jnp.int32)
               + (cib >= _LVL_CHUNK_BASE[4]).astype(jnp.int32))
        rf = rtab_v[pl.ds(cib * 16, 16)]
        cf = ctab_v[pl.ds(cib * 16, 16)]
        tbase = img * 4096 + lvl * 4 * 128
        kbase = img * 4096 + (20 + lvl) * 128
        lbase = img * 4096 + 25 * 128
        quads = []
        for off in (0, 128, 256, 384):
            quads.append((tab_v[pl.ds(tbase + off, 16)],
                          tab_v[pl.ds(tbase + off + 16, 16)]))
        key_lo = tab_v[pl.ds(kbase, 16)]
        key_hi = tab_v[pl.ds(kbase + 16, 16)]
        lab_lo = tab_v[pl.ds(lbase, 16)]
        lab_hi = tab_v[pl.ds(lbase + 16, 16)]
        best = jnp.full((16,), inf, jnp.float32)
        bpt = jnp.full((16,), -1, jnp.int32)
        bcat = jnp.zeros((16,), jnp.int32)
        for o in range(_NOBJ):
            half_idx = o % 16

            def pick(lo, hi):
                v = lo if o < 16 else hi
                s1 = lax.slice(v, (half_idx,), (half_idx + 1,))
                return jnp.broadcast_to(s1, (16,))

            p_l = pick(*quads[0])
            p_r = pick(*quads[1])
            p_t = pick(*quads[2])
            p_b = pick(*quads[3])
            key = pick(key_lo, key_hi)
            lab = pick(lab_lo, lab_hi)
            rect = (rf >= p_t) & (rf <= p_b) & (cf >= p_l) & (cf <= p_r)
            infv = jnp.full((16,), inf, jnp.float32)
            keyrect = jnp.where(rect, key, infv)
            upd = (keyrect <= best) & (keyrect < infv)
            best = jnp.where(upd, keyrect, best)
            bpt = jnp.where(upd, o, bpt)
            bcat = jnp.where(upd, lab.astype(jnp.int32), bcat)
        cat_st[pl.ds(t * 16, 16)] = bcat
        pt_st[pl.ds(t * 16, 16)] = bpt
        return carry

    lax.fori_loop(0, _CPW, chunk_body, 0)
    pltpu.sync_copy(cat_st, cat_hbm.at[pl.ds(base_chunk * 16, _CPW * 16)])
    pltpu.sync_copy(pt_st, pt_hbm.at[pl.ds(base_chunk * 16, _CPW * 16)])


def kernel(boxes, labels, masks):
    n, obj = masks.shape[0], masks.shape[1]
    boxes = jnp.asarray(boxes, dtype=jnp.float32)
    labels = jnp.asarray(labels, dtype=jnp.int32)
    masks = jnp.asarray(masks, dtype=jnp.float32)
    boxes_p = jnp.pad(boxes.transpose(0, 2, 1),
                      ((0, 0), (0, 4), (0, 128 - obj)))
    labels_p = jnp.pad(labels.reshape(n, 1, obj),
                       ((0, 0), (0, 0), (0, 128 - obj)))
    tab = pl.pallas_call(
        _stats_body,
        grid=(n, obj // _MPB),
        in_specs=[
            pl.BlockSpec((1, _MPB, _IMG, _IMG), lambda b, o: (b, o, 0, 0)),
            pl.BlockSpec((1, 8, 128), lambda b, o: (b, 0, 0)),
            pl.BlockSpec((1, 1, 128), lambda b, o: (b, 0, 0)),
        ],
        out_specs=pl.BlockSpec((1, 32, 128), lambda b, o: (b, 0, 0)),
        out_shape=jax.ShapeDtypeStruct((n, 32, 128), jnp.float32),
        scratch_shapes=[pltpu.VMEM((8, 128), jnp.int32)],
    )(masks, boxes_p, labels_p)
    tab_flat = tab.reshape(n * 32 * 128)
    mesh = plsc.VectorSubcoreMesh(core_axis_name="c", subcore_axis_name="s")
    sc = functools.partial(
        pl.kernel,
        mesh=mesh,
        out_type=[
            jax.ShapeDtypeStruct((_NOUT,), jnp.int32),
            jax.ShapeDtypeStruct((_NOUT,), jnp.int32),
        ],
        scratch_types=[
            pltpu.VMEM((n * 32 * 128,), jnp.float32),
            pltpu.VMEM((_NCELL,), jnp.float32),
            pltpu.VMEM((_NCELL,), jnp.float32),
            pltpu.VMEM((_CPW * 16,), jnp.int32),
            pltpu.VMEM((_CPW * 16,), jnp.int32),
        ],
    )(_sc_assign)
    cat_flat, pt_flat = sc(tab_flat, jnp.asarray(_RTAB), jnp.asarray(_CTAB))
    cat = cat_flat[:4 * _NCELL].reshape(n, _NCELL)
    pt = pt_flat[:4 * _NCELL].reshape(n, _NCELL)
    return cat, pt
